# Initial kernel scaffold; baseline (speedup 1.0000x reference)
import jax
import jax.numpy as jnp
from jax.experimental import pallas as pl


def kernel(predicted_edge_uvp, edge_index, face):
    raise NotImplementedError("write your pallas kernel here")


# speedup vs baseline: 2.9051x; 2.9051x over previous
import jax
import jax.numpy as jnp
from jax import lax
from jax.experimental import pallas as pl
from jax.experimental.pallas import tpu as pltpu
from jax.experimental.pallas import tpu_sc as plsc

N_NODES = 100000
N_EDGES = 1600000
N_CELLS = 200000

NC = 2
NS = 16
NW = NC * NS

CHUNK = 2000
EPT = N_EDGES // NW
E_CHUNKS = EPT // CHUNK
C_CHUNKS = N_CELLS // CHUNK
N_CHUNKS = N_NODES // CHUNK
LANES = 16

_mesh = plsc.VectorSubcoreMesh(core_axis_name="c", subcore_axis_name="s")
_params = pltpu.CompilerParams(use_tc_tiling_on_sc=False)


def _stage1_body(u_hbm, v_hbm, p_hbm, uvpf_hbm, send_hbm, recv_hbm,
                 f0_hbm, f1_hbm, f2_hbm, zeros_hbm,
                 u0_out, v0_out, p0_out, n0_out,
                 u1_out, v1_out, p1_out, n1_out,
                 cell0_out, cell1_out, cell2_out,
                 accu_sp, accv_sp, accp_sp, accn_sp,
                 su_v, sv_v, sp_v, ones_v, sidx_v, ridx_v,
                 f_v, ix_v, g_v, a0_v, a1_v, a2_v):
    cid = lax.axis_index("c")
    sid = lax.axis_index("s")
    wid = sid * NC + cid

    for t in range(4):
        ch = sid + NS * t
        @pl.when(ch < N_CHUNKS)
        def _():
            sl = pl.ds(ch * CHUNK, CHUNK)
            pltpu.sync_copy(zeros_hbm, accu_sp.at[sl])
            pltpu.sync_copy(zeros_hbm, accv_sp.at[sl])
            pltpu.sync_copy(zeros_hbm, accp_sp.at[sl])
            pltpu.sync_copy(zeros_hbm, accn_sp.at[sl])

    def _fill_ones(i, carry):
        ones_v[pl.ds(i * LANES, LANES)] = jnp.full((LANES,), 1.0, jnp.float32)
        return carry
    lax.fori_loop(0, CHUNK // LANES, _fill_ones, None)

    plsc.subcore_barrier()

    for j in range(E_CHUNKS):
        base = wid * EPT + j * CHUNK
        sl = pl.ds(base, CHUNK)
        pltpu.sync_copy(u_hbm.at[sl], su_v)
        pltpu.sync_copy(v_hbm.at[sl], sv_v)
        pltpu.sync_copy(p_hbm.at[sl], sp_v)
        pltpu.sync_copy(send_hbm.at[sl], sidx_v)
        pltpu.sync_copy(recv_hbm.at[sl], ridx_v)
        for ix in (sidx_v, ridx_v):
            pltpu.sync_copy(su_v, accu_sp.at[ix], add=True)
            pltpu.sync_copy(sv_v, accv_sp.at[ix], add=True)
            pltpu.sync_copy(sp_v, accp_sp.at[ix], add=True)
            pltpu.sync_copy(ones_v, accn_sp.at[ix], add=True)

    for t in range(4):
        ch = wid + NW * t
        @pl.when(ch < C_CHUNKS)
        def _():
            base = ch * CHUNK
            accs = (a0_v, a1_v, a2_v)
            for k, fk in enumerate((f0_hbm, f1_hbm, f2_hbm)):
                pltpu.sync_copy(fk.at[pl.ds(base, CHUNK)], f_v)

                def _mk_idx(i, c):
                    sl = pl.ds(i * LANES, LANES)
                    ix_v[sl] = f_v[sl] * 3
                    return c
                lax.fori_loop(0, CHUNK // LANES, _mk_idx, None)

                for m in range(3):
                    if k == 0:
                        pltpu.sync_copy(uvpf_hbm.at[ix_v], accs[m])
                    else:
                        pltpu.sync_copy(uvpf_hbm.at[ix_v], g_v)

                        def _acc(i, c, am=accs[m]):
                            sl = pl.ds(i * LANES, LANES)
                            am[sl] = am[sl] + g_v[sl]
                            return c
                        lax.fori_loop(0, CHUNK // LANES, _acc, None)
                    if m < 2:
                        def _inc(i, c):
                            sl = pl.ds(i * LANES, LANES)
                            ix_v[sl] = ix_v[sl] + 1
                            return c
                        lax.fori_loop(0, CHUNK // LANES, _inc, None)

            def _scale(i, c):
                sl = pl.ds(i * LANES, LANES)
                a0_v[sl] = a0_v[sl] / 3.0
                a1_v[sl] = a1_v[sl] / 3.0
                a2_v[sl] = a2_v[sl] / 3.0
                return c
            lax.fori_loop(0, CHUNK // LANES, _scale, None)

            pltpu.sync_copy(a0_v, cell0_out.at[pl.ds(base, CHUNK)])
            pltpu.sync_copy(a1_v, cell1_out.at[pl.ds(base, CHUNK)])
            pltpu.sync_copy(a2_v, cell2_out.at[pl.ds(base, CHUNK)])

    plsc.subcore_barrier()
    for t in range(4):
        ch = sid + NS * t
        @pl.when(ch < N_CHUNKS)
        def _():
            sl = pl.ds(ch * CHUNK, CHUNK)
            @pl.when(cid == 0)
            def _():
                pltpu.sync_copy(accu_sp.at[sl], u0_out.at[sl])
                pltpu.sync_copy(accv_sp.at[sl], v0_out.at[sl])
                pltpu.sync_copy(accp_sp.at[sl], p0_out.at[sl])
                pltpu.sync_copy(accn_sp.at[sl], n0_out.at[sl])
            @pl.when(cid == 1)
            def _():
                pltpu.sync_copy(accu_sp.at[sl], u1_out.at[sl])
                pltpu.sync_copy(accv_sp.at[sl], v1_out.at[sl])
                pltpu.sync_copy(accp_sp.at[sl], p1_out.at[sl])
                pltpu.sync_copy(accn_sp.at[sl], n1_out.at[sl])


_N1 = jax.ShapeDtypeStruct((N_NODES,), jnp.float32)
_C1 = jax.ShapeDtypeStruct((N_CELLS,), jnp.float32)

_stage1 = pl.kernel(
    _stage1_body,
    out_type=(_N1, _N1, _N1, _N1, _N1, _N1, _N1, _N1, _C1, _C1, _C1),
    mesh=_mesh,
    compiler_params=_params,
    scratch_types=(
        [pltpu.VMEM_SHARED((N_NODES,), jnp.float32)] * 4
        + [pltpu.VMEM((CHUNK,), jnp.float32)] * 4
        + [pltpu.VMEM((CHUNK,), jnp.int32)] * 4
        + [pltpu.VMEM((CHUNK,), jnp.float32)] * 4
    ),
)


def _stage2_body(u0_hbm, v0_hbm, p0_hbm, n0_hbm, u1_hbm, v1_hbm, p1_hbm, n1_hbm,
                 nu_out, nv_out, np_out,
                 a_v, b_v, c0_v, c1_v, den_v, o_v):
    cid = lax.axis_index("c")
    sid = lax.axis_index("s")
    wid = sid * NC + cid

    for t in range(2):
        ch = wid + NW * t
        @pl.when(ch < N_CHUNKS)
        def _():
            sl = pl.ds(ch * CHUNK, CHUNK)
            pltpu.sync_copy(n0_hbm.at[sl], c0_v)
            pltpu.sync_copy(n1_hbm.at[sl], c1_v)

            def _den(i, c):
                s = pl.ds(i * LANES, LANES)
                den_v[s] = jnp.maximum(c0_v[s] + c1_v[s], 1.0)
                return c
            lax.fori_loop(0, CHUNK // LANES, _den, None)

            for (x0, x1, out) in ((u0_hbm, u1_hbm, nu_out),
                                  (v0_hbm, v1_hbm, nv_out),
                                  (p0_hbm, p1_hbm, np_out)):
                pltpu.sync_copy(x0.at[sl], a_v)
                pltpu.sync_copy(x1.at[sl], b_v)

                def _norm(i, c):
                    s = pl.ds(i * LANES, LANES)
                    o_v[s] = (a_v[s] + b_v[s]) / den_v[s]
                    return c
                lax.fori_loop(0, CHUNK // LANES, _norm, None)
                pltpu.sync_copy(o_v, out.at[sl])


_stage2 = pl.kernel(
    _stage2_body,
    out_type=(_N1, _N1, _N1),
    mesh=_mesh,
    compiler_params=_params,
    scratch_types=[pltpu.VMEM((CHUNK,), jnp.float32)] * 6,
)


@jax.jit
def kernel(predicted_edge_uvp, edge_index, face):
    senders = edge_index[0]
    receivers = edge_index[1]
    f0, f1, f2 = face[0], face[1], face[2]
    uvp_t = predicted_edge_uvp.T
    uvp_flat = predicted_edge_uvp.reshape(N_EDGES * 3)
    zeros = jnp.zeros((CHUNK,), jnp.float32)
    u0, v0, p0, n0, u1, v1, p1, n1, c0, c1, c2 = _stage1(
        uvp_t[0], uvp_t[1], uvp_t[2], uvp_flat, senders, receivers,
        f0, f1, f2, zeros)
    nu, nv, np_ = _stage2(u0, v0, p0, n0, u1, v1, p1, n1)
    node_uvp = jnp.stack([nu, nv, np_], axis=1)
    cell_uvp = jnp.stack([c0, c1, c2], axis=1)
    return node_uvp, cell_uvp
